# baseline (speedup 1.0000x reference)
import jax
import jax.numpy as jnp
from jax import lax
from jax.experimental import pallas as pl
from jax.experimental.pallas import tpu as pltpu
from jax.experimental.pallas import tpu_sc as plsc

N_NODES = 10000
D = 128
E = 320000
NC = 2
NS = 16
NW = NC * NS
EPW = E // NW
CHUNK = 64
NBUF = 4
NPHASE = 2
PROWS = 40
PCHUNKS = 2 * PROWS
NCHUNKS = NPHASE * PCHUNKS
PAD = NCHUNKS * CHUNK - EPW
NG = PCHUNKS // NBUF
N_ACC = N_NODES + 8
RPW = 624
ZROWS = 8


def _sc_aggregate(x, src, dst):
    mesh = plsc.VectorSubcoreMesh(core_axis_name="c", subcore_axis_name="s")

    def body(x_hbm, src_hbm, dst_hbm, out_hbm,
             gb0, gb1, gb2, gb3, src_v, dst_v, zbuf, accum,
             zsem, isem, gsem, ssem):
        gbufs = (gb0, gb1, gb2, gb3)
        cid = lax.axis_index("c")
        sid = lax.axis_index("s")
        wid = sid * NC + cid

        i0 = pltpu.async_copy(src_hbm.at[wid, 0], src_v, isem)
        i1 = pltpu.async_copy(dst_hbm.at[wid, 0], dst_v, isem)

        @pl.loop(0, ZROWS)
        def _(r):
            @pl.loop(0, D // 16)
            def _(c):
                zbuf[r, pl.ds(c * 16, 16)] = jnp.zeros((16,), jnp.float32)

        row0 = sid * RPW
        descs = [pltpu.async_copy(zbuf, accum.at[pl.ds(row0 + i * ZROWS, ZROWS)],
                                  zsem)
                 for i in range(RPW // ZROWS)]

        @pl.when(sid == NS - 1)
        def _():
            d1 = pltpu.async_copy(zbuf, accum.at[pl.ds(row0 + RPW, ZROWS)], zsem)
            d2 = pltpu.async_copy(zbuf, accum.at[pl.ds(row0 + RPW + ZROWS,
                                                       ZROWS)], zsem)
            d1.wait()
            d2.wait()

        for d in descs:
            d.wait()
        i0.wait()
        i1.wait()
        plsc.subcore_barrier()

        def src_idx(g, b):
            return src_v.at[g * (NBUF // 2) + (b >> 1),
                            pl.ds((b & 1) * CHUNK, CHUNK)]

        def dst_idx(g, b):
            return dst_v.at[g * (NBUF // 2) + (b >> 1),
                            pl.ds((b & 1) * CHUNK, CHUNK)]

        for phase in range(NPHASE):
            for b in range(NBUF):
                pltpu.async_copy(x_hbm.at[src_idx(0, b)], gbufs[b], gsem.at[b])

            @pl.loop(0, NG)
            def _(g):
                for b in range(NBUF):
                    pltpu.make_async_copy(x_hbm.at[src_idx(0, b)], gbufs[b],
                                          gsem.at[b]).wait()
                for b in range(NBUF):
                    pltpu.async_copy(gbufs[b], accum.at[dst_idx(g, b)],
                                     ssem.at[b], add=True)

                @pl.when(g < NG - 1)
                def _():
                    for b in range(NBUF):
                        pltpu.make_async_copy(gbufs[b], accum.at[dst_idx(0, b)],
                                              ssem.at[b]).wait()
                    for b in range(NBUF):
                        pltpu.async_copy(x_hbm.at[src_idx(g + 1, b)],
                                         gbufs[b], gsem.at[b])

            for b in range(NBUF):
                pltpu.make_async_copy(gbufs[b], accum.at[dst_idx(0, b)],
                                      ssem.at[b]).wait()
            if phase < NPHASE - 1:
                j0 = pltpu.async_copy(src_hbm.at[wid, phase + 1], src_v, isem)
                j1 = pltpu.async_copy(dst_hbm.at[wid, phase + 1], dst_v, isem)
                j0.wait()
                j1.wait()

        plsc.subcore_barrier()

        @pl.when(sid < NS - 1)
        def _():
            pltpu.sync_copy(accum.at[pl.ds(row0, RPW)],
                            out_hbm.at[cid, pl.ds(row0, RPW)])

        @pl.when(sid == NS - 1)
        def _():
            pltpu.sync_copy(accum.at[pl.ds(row0, RPW + 2 * ZROWS)],
                            out_hbm.at[cid, pl.ds(row0, RPW + 2 * ZROWS)])

    f = pl.kernel(
        body,
        out_type=jax.ShapeDtypeStruct((NC, N_NODES, D), jnp.float32),
        mesh=mesh,
        scratch_types=[
            pltpu.VMEM((CHUNK, D), jnp.float32),
            pltpu.VMEM((CHUNK, D), jnp.float32),
            pltpu.VMEM((CHUNK, D), jnp.float32),
            pltpu.VMEM((CHUNK, D), jnp.float32),
            pltpu.VMEM((PROWS, 128), jnp.int32),
            pltpu.VMEM((PROWS, 128), jnp.int32),
            pltpu.VMEM((ZROWS, D), jnp.float32),
            pltpu.VMEM_SHARED((N_ACC, D), jnp.float32),
            pltpu.SemaphoreType.DMA,
            pltpu.SemaphoreType.DMA,
            pltpu.SemaphoreType.DMA((NBUF,)),
            pltpu.SemaphoreType.DMA((NBUF,)),
        ],
    )
    return f(x, src, dst)


def _tc_mlp(x, p0, p1, W1, b1, W2, b2):
    BR = 2000
    dn = (((1,), (0,)), ((), ()))

    def body(x_ref, p0_ref, p1_ref, w1_ref, b1_ref, w2_ref, b2_ref, o_ref):
        h = x_ref[...] + p0_ref[...] + p1_ref[...]
        h1 = lax.dot_general(h, w1_ref[...], dn,
                             precision=lax.Precision.HIGHEST,
                             preferred_element_type=jnp.float32) + b1_ref[...]
        h1 = jnp.maximum(h1, 0.0)
        o_ref[...] = lax.dot_general(h1, w2_ref[...], dn,
                                     precision=lax.Precision.HIGHEST,
                                     preferred_element_type=jnp.float32) + b2_ref[...]

    return pl.pallas_call(
        body,
        grid=(N_NODES // BR,),
        in_specs=[
            pl.BlockSpec((BR, D), lambda i: (i, 0)),
            pl.BlockSpec((BR, D), lambda i: (i, 0)),
            pl.BlockSpec((BR, D), lambda i: (i, 0)),
            pl.BlockSpec((D, D), lambda i: (0, 0)),
            pl.BlockSpec((1, D), lambda i: (0, 0)),
            pl.BlockSpec((D, D), lambda i: (0, 0)),
            pl.BlockSpec((1, D), lambda i: (0, 0)),
        ],
        out_specs=pl.BlockSpec((BR, D), lambda i: (i, 0)),
        out_shape=jax.ShapeDtypeStruct((N_NODES, D), jnp.float32),
    )(x, p0, p1, W1, b1.reshape(1, D), W2, b2.reshape(1, D))


def kernel(x, edge_index, W1, b1, W2, b2):
    src_pad = jnp.zeros((NW, PAD), jnp.int32)
    dst_pad = jnp.full((NW, PAD), N_NODES, jnp.int32)
    src = jnp.concatenate([edge_index[0].reshape(NW, EPW), src_pad],
                          axis=1).reshape(NW, NPHASE, PROWS, 128)
    dst = jnp.concatenate([edge_index[1].reshape(NW, EPW), dst_pad],
                          axis=1).reshape(NW, NPHASE, PROWS, 128)
    p = _sc_aggregate(x, src, dst)
    return _tc_mlp(x, p[0], p[1], W1, b1, W2, b2)


# speedup vs baseline: 1.0578x; 1.0578x over previous
import jax
import jax.numpy as jnp
from jax import lax
from jax.experimental import pallas as pl
from jax.experimental.pallas import tpu as pltpu
from jax.experimental.pallas import tpu_sc as plsc

N_NODES = 10000
D = 128
E = 320000
NC = 2
NS = 16
NW = NC * NS
EPW = E // NW
CHUNK = 64
NBUF = 4
NPHASE = 2
PROWS = 40
PCHUNKS = 2 * PROWS
NCHUNKS = NPHASE * PCHUNKS
PAD = NCHUNKS * CHUNK - EPW
NG = PCHUNKS // NBUF
N_ACC = N_NODES + 8
RPW = 624
ZROWS = 8


def _sc_aggregate(x, src, dst):
    mesh = plsc.VectorSubcoreMesh(core_axis_name="c", subcore_axis_name="s")

    def body(x_hbm, src_hbm, dst_hbm, out_hbm,
             gb0, gb1, gb2, gb3, src_v, dst_v, zbuf, accum,
             zsem, isem, gsem, ssem):
        gbufs = (gb0, gb1, gb2, gb3)
        cid = lax.axis_index("c")
        sid = lax.axis_index("s")
        wid = sid * NC + cid

        i0 = pltpu.async_copy(src_hbm.at[wid, 0], src_v, isem)
        i1 = pltpu.async_copy(dst_hbm.at[wid, 0], dst_v, isem)

        @pl.loop(0, ZROWS)
        def _(r):
            @pl.loop(0, D // 16)
            def _(c):
                zbuf[r, pl.ds(c * 16, 16)] = jnp.zeros((16,), jnp.float32)

        row0 = sid * RPW
        descs = [pltpu.async_copy(zbuf, accum.at[pl.ds(row0 + i * ZROWS, ZROWS)],
                                  zsem)
                 for i in range(RPW // ZROWS)]

        @pl.when(sid == NS - 1)
        def _():
            d1 = pltpu.async_copy(zbuf, accum.at[pl.ds(row0 + RPW, ZROWS)], zsem)
            d2 = pltpu.async_copy(zbuf, accum.at[pl.ds(row0 + RPW + ZROWS,
                                                       ZROWS)], zsem)
            d1.wait()
            d2.wait()

        for d in descs:
            d.wait()
        i0.wait()
        i1.wait()
        plsc.subcore_barrier()

        def src_idx(g, b):
            return src_v.at[g * (NBUF // 2) + (b >> 1),
                            pl.ds((b & 1) * CHUNK, CHUNK)]

        def dst_idx(g, b):
            return dst_v.at[g * (NBUF // 2) + (b >> 1),
                            pl.ds((b & 1) * CHUNK, CHUNK)]

        LAG = 2
        for phase in range(NPHASE):
            for b in range(LAG):
                pltpu.async_copy(x_hbm.at[src_idx(0, b)], gbufs[b], gsem.at[b])

            @pl.loop(0, NG)
            def _(g):
                for b in range(NBUF):
                    pltpu.make_async_copy(x_hbm.at[src_idx(0, b)], gbufs[b],
                                          gsem.at[b]).wait()
                    pltpu.async_copy(gbufs[b], accum.at[dst_idx(g, b)],
                                     ssem, add=True)
                    tgt = (b + LAG) % NBUF
                    if b < LAG:
                        @pl.when(g > 0)
                        def _():
                            pltpu.make_async_copy(gbufs[b],
                                                  accum.at[dst_idx(0, b)],
                                                  ssem).wait()

                        pltpu.async_copy(x_hbm.at[src_idx(g, b + LAG)],
                                         gbufs[tgt], gsem.at[tgt])
                    else:
                        pltpu.make_async_copy(gbufs[b], accum.at[dst_idx(0, b)],
                                              ssem).wait()

                        @pl.when(g < NG - 1)
                        def _():
                            pltpu.async_copy(x_hbm.at[src_idx(g, b + LAG)],
                                             gbufs[tgt], gsem.at[tgt])

            for b in range(LAG):
                pltpu.make_async_copy(gbufs[b], accum.at[dst_idx(0, b)],
                                      ssem).wait()
            if phase < NPHASE - 1:
                j0 = pltpu.async_copy(src_hbm.at[wid, phase + 1], src_v, isem)
                j1 = pltpu.async_copy(dst_hbm.at[wid, phase + 1], dst_v, isem)
                j0.wait()
                j1.wait()

        plsc.subcore_barrier()

        @pl.when(sid < NS - 1)
        def _():
            pltpu.sync_copy(accum.at[pl.ds(row0, RPW)],
                            out_hbm.at[cid, pl.ds(row0, RPW)])

        @pl.when(sid == NS - 1)
        def _():
            pltpu.sync_copy(accum.at[pl.ds(row0, RPW + 2 * ZROWS)],
                            out_hbm.at[cid, pl.ds(row0, RPW + 2 * ZROWS)])

    f = pl.kernel(
        body,
        out_type=jax.ShapeDtypeStruct((NC, N_NODES, D), jnp.float32),
        mesh=mesh,
        scratch_types=[
            pltpu.VMEM((CHUNK, D), jnp.float32),
            pltpu.VMEM((CHUNK, D), jnp.float32),
            pltpu.VMEM((CHUNK, D), jnp.float32),
            pltpu.VMEM((CHUNK, D), jnp.float32),
            pltpu.VMEM((PROWS, 128), jnp.int32),
            pltpu.VMEM((PROWS, 128), jnp.int32),
            pltpu.VMEM((ZROWS, D), jnp.float32),
            pltpu.VMEM_SHARED((N_ACC, D), jnp.float32),
            pltpu.SemaphoreType.DMA,
            pltpu.SemaphoreType.DMA,
            pltpu.SemaphoreType.DMA((NBUF,)),
            pltpu.SemaphoreType.DMA,
        ],
    )
    return f(x, src, dst)


def _tc_mlp(x, p0, p1, W1, b1, W2, b2):
    BR = 2000
    dn = (((1,), (0,)), ((), ()))

    def body(x_ref, p0_ref, p1_ref, w1_ref, b1_ref, w2_ref, b2_ref, o_ref):
        h = x_ref[...] + p0_ref[...] + p1_ref[...]
        h1 = lax.dot_general(h, w1_ref[...], dn,
                             precision=lax.Precision.HIGHEST,
                             preferred_element_type=jnp.float32) + b1_ref[...]
        h1 = jnp.maximum(h1, 0.0)
        o_ref[...] = lax.dot_general(h1, w2_ref[...], dn,
                                     precision=lax.Precision.HIGHEST,
                                     preferred_element_type=jnp.float32) + b2_ref[...]

    return pl.pallas_call(
        body,
        grid=(N_NODES // BR,),
        in_specs=[
            pl.BlockSpec((BR, D), lambda i: (i, 0)),
            pl.BlockSpec((BR, D), lambda i: (i, 0)),
            pl.BlockSpec((BR, D), lambda i: (i, 0)),
            pl.BlockSpec((D, D), lambda i: (0, 0)),
            pl.BlockSpec((1, D), lambda i: (0, 0)),
            pl.BlockSpec((D, D), lambda i: (0, 0)),
            pl.BlockSpec((1, D), lambda i: (0, 0)),
        ],
        out_specs=pl.BlockSpec((BR, D), lambda i: (i, 0)),
        out_shape=jax.ShapeDtypeStruct((N_NODES, D), jnp.float32),
    )(x, p0, p1, W1, b1.reshape(1, D), W2, b2.reshape(1, D))


def kernel(x, edge_index, W1, b1, W2, b2):
    src_pad = jnp.zeros((NW, PAD), jnp.int32)
    dst_pad = jnp.full((NW, PAD), N_NODES, jnp.int32)
    src = jnp.concatenate([edge_index[0].reshape(NW, EPW), src_pad],
                          axis=1).reshape(NW, NPHASE, PROWS, 128)
    dst = jnp.concatenate([edge_index[1].reshape(NW, EPW), dst_pad],
                          axis=1).reshape(NW, NPHASE, PROWS, 128)
    p = _sc_aggregate(x, src, dst)
    return _tc_mlp(x, p[0], p[1], W1, b1, W2, b2)


# speedup vs baseline: 1.1156x; 1.0547x over previous
import jax
import jax.numpy as jnp
from jax import lax
from jax.experimental import pallas as pl
from jax.experimental.pallas import tpu as pltpu
from jax.experimental.pallas import tpu_sc as plsc

N_NODES = 10000
D = 128
E = 320000
NC = 2
NS = 16
NW = NC * NS
EPW = E // NW
CHUNK = 64
NBUF = 4
NPHASE = 2
PROWS = 40
PCHUNKS = 2 * PROWS
NCHUNKS = NPHASE * PCHUNKS
PAD = NCHUNKS * CHUNK - EPW
NG = PCHUNKS // NBUF
N_ACC = N_NODES + 8
RPW = 624
ZROWS = 8


def _sc_aggregate(x, src, dst):
    mesh = plsc.VectorSubcoreMesh(core_axis_name="c", subcore_axis_name="s")

    def body(x_hbm, src_hbm, dst_hbm, out_hbm,
             gb0, gb1, gb2, gb3, src_v, dst_v, zbuf, accum,
             zsem, isem, gsem, ssem):
        gbufs = (gb0, gb1, gb2, gb3)
        cid = lax.axis_index("c")
        sid = lax.axis_index("s")
        wid = sid * NC + cid

        i0 = pltpu.async_copy(src_hbm.at[wid, 0], src_v, isem)
        i1 = pltpu.async_copy(dst_hbm.at[wid, 0], dst_v, isem)

        @pl.loop(0, ZROWS)
        def _(r):
            @pl.loop(0, D // 16)
            def _(c):
                zbuf[r, pl.ds(c * 16, 16)] = jnp.zeros((16,), jnp.float32)

        row0 = sid * RPW
        descs = [pltpu.async_copy(zbuf, accum.at[pl.ds(row0 + i * ZROWS, ZROWS)],
                                  zsem)
                 for i in range(RPW // ZROWS)]

        @pl.when(sid == NS - 1)
        def _():
            d1 = pltpu.async_copy(zbuf, accum.at[pl.ds(row0 + RPW, ZROWS)], zsem)
            d2 = pltpu.async_copy(zbuf, accum.at[pl.ds(row0 + RPW + ZROWS,
                                                       ZROWS)], zsem)
            d1.wait()
            d2.wait()

        for d in descs:
            d.wait()
        i0.wait()
        i1.wait()
        plsc.subcore_barrier()

        def src_idx(g, b):
            return src_v.at[g * (NBUF // 2) + (b >> 1),
                            pl.ds((b & 1) * CHUNK, CHUNK)]

        def dst_idx(g, b):
            return dst_v.at[g * (NBUF // 2) + (b >> 1),
                            pl.ds((b & 1) * CHUNK, CHUNK)]

        for phase in range(NPHASE):
            for b in range(NBUF):
                pltpu.async_copy(x_hbm.at[src_idx(0, b)], gbufs[b], gsem.at[b])

            @pl.loop(0, NG)
            def _(g):
                for b in range(NBUF):
                    pltpu.make_async_copy(x_hbm.at[src_idx(0, b)], gbufs[b],
                                          gsem.at[b]).wait()
                    pltpu.async_copy(gbufs[b], accum.at[dst_idx(g, b)],
                                     ssem.at[b], add=True)

                    @pl.when(g < NG - 1)
                    def _():
                        pltpu.make_async_copy(gbufs[b], accum.at[dst_idx(0, b)],
                                              ssem.at[b]).wait()
                        pltpu.async_copy(x_hbm.at[src_idx(g + 1, b)],
                                         gbufs[b], gsem.at[b])

            for b in range(NBUF):
                pltpu.make_async_copy(gbufs[b], accum.at[dst_idx(0, b)],
                                      ssem.at[b]).wait()
            if phase < NPHASE - 1:
                j0 = pltpu.async_copy(src_hbm.at[wid, phase + 1], src_v, isem)
                j1 = pltpu.async_copy(dst_hbm.at[wid, phase + 1], dst_v, isem)
                j0.wait()
                j1.wait()

        plsc.subcore_barrier()

        @pl.when(sid < NS - 1)
        def _():
            pltpu.sync_copy(accum.at[pl.ds(row0, RPW)],
                            out_hbm.at[cid, pl.ds(row0, RPW)])

        @pl.when(sid == NS - 1)
        def _():
            pltpu.sync_copy(accum.at[pl.ds(row0, RPW + 2 * ZROWS)],
                            out_hbm.at[cid, pl.ds(row0, RPW + 2 * ZROWS)])

    f = pl.kernel(
        body,
        out_type=jax.ShapeDtypeStruct((NC, N_NODES, D), jnp.float32),
        mesh=mesh,
        scratch_types=[
            pltpu.VMEM((CHUNK, D), jnp.float32),
            pltpu.VMEM((CHUNK, D), jnp.float32),
            pltpu.VMEM((CHUNK, D), jnp.float32),
            pltpu.VMEM((CHUNK, D), jnp.float32),
            pltpu.VMEM((PROWS, 128), jnp.int32),
            pltpu.VMEM((PROWS, 128), jnp.int32),
            pltpu.VMEM((ZROWS, D), jnp.float32),
            pltpu.VMEM_SHARED((N_ACC, D), jnp.float32),
            pltpu.SemaphoreType.DMA,
            pltpu.SemaphoreType.DMA,
            pltpu.SemaphoreType.DMA((NBUF,)),
            pltpu.SemaphoreType.DMA((NBUF,)),
        ],
    )
    return f(x, src, dst)


def _tc_mlp(x, p0, p1, W1, b1, W2, b2):
    BR = 2000
    dn = (((1,), (0,)), ((), ()))

    def body(x_ref, p0_ref, p1_ref, w1_ref, b1_ref, w2_ref, b2_ref, o_ref):
        h = x_ref[...] + p0_ref[...] + p1_ref[...]
        h1 = lax.dot_general(h, w1_ref[...], dn,
                             precision=lax.Precision.HIGHEST,
                             preferred_element_type=jnp.float32) + b1_ref[...]
        h1 = jnp.maximum(h1, 0.0)
        o_ref[...] = lax.dot_general(h1, w2_ref[...], dn,
                                     precision=lax.Precision.HIGHEST,
                                     preferred_element_type=jnp.float32) + b2_ref[...]

    return pl.pallas_call(
        body,
        grid=(N_NODES // BR,),
        in_specs=[
            pl.BlockSpec((BR, D), lambda i: (i, 0)),
            pl.BlockSpec((BR, D), lambda i: (i, 0)),
            pl.BlockSpec((BR, D), lambda i: (i, 0)),
            pl.BlockSpec((D, D), lambda i: (0, 0)),
            pl.BlockSpec((1, D), lambda i: (0, 0)),
            pl.BlockSpec((D, D), lambda i: (0, 0)),
            pl.BlockSpec((1, D), lambda i: (0, 0)),
        ],
        out_specs=pl.BlockSpec((BR, D), lambda i: (i, 0)),
        out_shape=jax.ShapeDtypeStruct((N_NODES, D), jnp.float32),
    )(x, p0, p1, W1, b1.reshape(1, D), W2, b2.reshape(1, D))


def kernel(x, edge_index, W1, b1, W2, b2):
    src_pad = jnp.zeros((NW, PAD), jnp.int32)
    dst_pad = jnp.full((NW, PAD), N_NODES, jnp.int32)
    src = jnp.concatenate([edge_index[0].reshape(NW, EPW), src_pad],
                          axis=1).reshape(NW, NPHASE, PROWS, 128)
    dst = jnp.concatenate([edge_index[1].reshape(NW, EPW), dst_pad],
                          axis=1).reshape(NW, NPHASE, PROWS, 128)
    p = _sc_aggregate(x, src, dst)
    return _tc_mlp(x, p[0], p[1], W1, b1, W2, b2)
